# Initial kernel scaffold; baseline (speedup 1.0000x reference)
import jax
import jax.numpy as jnp
from jax.experimental import pallas as pl


def kernel(x, W_enc, W_dec, b_dec):
    raise NotImplementedError("write your pallas kernel here")


# speedup vs baseline: 32.0847x; 32.0847x over previous
import functools

import jax
import jax.numpy as jnp
from jax.experimental import pallas as pl
from jax.experimental.pallas import tpu as pltpu

_N = 2048
_D = 768
_F = 16384
_KTOT = 32 * _N
_L1C = 1e-4

_EB = 1024
_DB = 512
_SB = 1024
_NB = 8
_R = 11


def _prep_body(x_ref, xn_ref, mu_ref, sd_ref):
    x = x_ref[...]
    mu = jnp.mean(x, axis=1, keepdims=True)
    xc = x - mu
    m2 = jnp.mean(xc, axis=1, keepdims=True)
    var = jnp.sum((xc - m2) * (xc - m2), axis=1, keepdims=True) / (_D - 1)
    sd = jnp.sqrt(var)
    xn_ref[...] = xc / (sd + 1e-5)
    mu_ref[...] = jnp.broadcast_to(mu, (_N, 128))
    sd_ref[...] = jnp.broadcast_to(sd, (_N, 128))


def _enc_body(xn_ref, w_ref, b_ref, acts_ref):
    xc = xn_ref[...] - b_ref[...]
    acts = jax.lax.dot_general(xc, w_ref[...], (((1,), (0,)), ((), ())),
                               preferred_element_type=jnp.float32)
    acts_ref[...] = jnp.maximum(acts, 0.0)


def _sel_body(acts_ref, tau_ref, sm, cnt):
    r = pl.program_id(0)
    t = pl.program_id(1)
    nt = pl.num_programs(1)

    @pl.when((r == 0) & (t == 0))
    def _():
        sm[0] = 0
        sm[1] = 0x7F800000

    @pl.when(t == 0)
    def _():
        for i in range(_NB):
            cnt[i] = 0

    lo = sm[0]
    hi = sm[1]
    step = jnp.maximum(jax.lax.div(hi - lo, _NB + 1), 1)
    bits = jax.lax.bitcast_convert_type(acts_ref[...], jnp.int32)
    bounds = []
    for i in range(_NB):
        b = lo + step * (i + 1)
        bounds.append(b)
        cnt[i] = cnt[i] + jnp.sum((bits >= b).astype(jnp.int32))

    @pl.when(t == nt - 1)
    def _():
        nlo = lo
        nhi = hi
        for i in range(_NB):
            ok = cnt[i] >= _KTOT
            nlo = jnp.where(ok, jnp.maximum(nlo, bounds[i]), nlo)
            nhi = jnp.where(ok, nhi, jnp.minimum(nhi, bounds[i]))
        sm[0] = nlo
        sm[1] = nhi
        tau_ref[...] = jnp.full((8, 128), nlo, jnp.int32)


def _dec_body(tau_ref, acts_ref, w_ref, xn_ref, mu_ref, sd_ref, b_ref,
              atk_ref, sae_ref, l1p_ref, l0p_ref, l2p_ref, acc):
    t = pl.program_id(0)
    nt = pl.num_programs(0)
    tau = tau_ref[0]
    a = acts_ref[...]
    m = a >= tau
    atk = jnp.where(m, a, 0.0)
    atk_ref[...] = atk
    l1p_ref[...] = jnp.full((1, 1, 128), jnp.sum(atk), jnp.float32)
    l0p_ref[...] = jnp.full((1, 1, 128), jnp.sum(m.astype(jnp.float32)),
                            jnp.float32)

    @pl.when(t == 0)
    def _():
        acc[...] = jnp.zeros_like(acc)

    acc[...] += jax.lax.dot_general(atk, w_ref[...], (((1,), (0,)), ((), ())),
                                    preferred_element_type=jnp.float32)

    @pl.when(t == nt - 1)
    def _():
        xr = acc[...] + b_ref[...]
        xn = xn_ref[...]
        d = xr - xn
        l2p_ref[...] = jnp.full((1, 128), jnp.sum(d * d), jnp.float32)
        sae_ref[...] = xr * sd_ref[...][:, 0:1] + mu_ref[...][:, 0:1]


_INTERP = False


@jax.jit
def kernel(x, W_enc, W_dec, b_dec):
    interp = _INTERP
    f32 = jnp.float32

    xn, mu, sd = pl.pallas_call(
        _prep_body,
        out_shape=[
            jax.ShapeDtypeStruct((_N, _D), f32),
            jax.ShapeDtypeStruct((_N, 128), f32),
            jax.ShapeDtypeStruct((_N, 128), f32),
        ],
        interpret=interp,
    )(x)

    n_eb = _F // _EB
    acts = pl.pallas_call(
        _enc_body,
        grid=(n_eb,),
        in_specs=[
            pl.BlockSpec((_N, _D), lambda i: (0, 0)),
            pl.BlockSpec((_D, _EB), lambda i: (0, i)),
            pl.BlockSpec((_D,), lambda i: (0,)),
        ],
        out_specs=pl.BlockSpec((_N, _EB), lambda i: (0, i)),
        out_shape=jax.ShapeDtypeStruct((_N, _F), f32),
        interpret=interp,
    )(xn, W_enc, b_dec)

    n_sb = _F // _SB
    taub = pl.pallas_call(
        _sel_body,
        grid=(_R, n_sb),
        in_specs=[pl.BlockSpec((_N, _SB), lambda r, t: (0, t))],
        out_specs=pl.BlockSpec((8, 128), lambda r, t: (0, 0)),
        out_shape=jax.ShapeDtypeStruct((8, 128), jnp.int32),
        scratch_shapes=[
            pltpu.SMEM((2,), jnp.int32),
            pltpu.SMEM((_NB,), jnp.int32),
        ],
        interpret=interp,
    )(acts)
    tau = jax.lax.bitcast_convert_type(taub[0, 0], f32)

    n_db = _F // _DB
    atk, sae, l1p, l0p, l2p = pl.pallas_call(
        _dec_body,
        grid=(n_db,),
        in_specs=[
            pl.BlockSpec(memory_space=pltpu.SMEM),
            pl.BlockSpec((_N, _DB), lambda i: (0, i)),
            pl.BlockSpec((_DB, _D), lambda i: (i, 0)),
            pl.BlockSpec((_N, _D), lambda i: (0, 0)),
            pl.BlockSpec((_N, 128), lambda i: (0, 0)),
            pl.BlockSpec((_N, 128), lambda i: (0, 0)),
            pl.BlockSpec((_D,), lambda i: (0,)),
        ],
        out_specs=[
            pl.BlockSpec((_N, _DB), lambda i: (0, i)),
            pl.BlockSpec((_N, _D), lambda i: (0, 0)),
            pl.BlockSpec((1, 1, 128), lambda i: (i, 0, 0)),
            pl.BlockSpec((1, 1, 128), lambda i: (i, 0, 0)),
            pl.BlockSpec((1, 128), lambda i: (0, 0)),
        ],
        out_shape=[
            jax.ShapeDtypeStruct((_N, _F), f32),
            jax.ShapeDtypeStruct((_N, _D), f32),
            jax.ShapeDtypeStruct((n_db, 1, 128), f32),
            jax.ShapeDtypeStruct((n_db, 1, 128), f32),
            jax.ShapeDtypeStruct((1, 128), f32),
        ],
        scratch_shapes=[pltpu.VMEM((_N, _D), f32)],
        interpret=interp,
    )(jnp.reshape(tau, (1,)), acts, W_dec, xn, mu, sd, b_dec)

    l1n = jnp.sum(l1p[:, 0, 0]) / _N
    l0n = jnp.sum(l0p[:, 0, 0]) / _N
    l2 = l2p[0, 0] / (_N * _D)
    loss = l2 + jnp.float32(0.0)
    return sae, atk, loss, l2, _L1C * l1n, l0n, l1n


# speedup vs baseline: 58.6476x; 1.8279x over previous
import functools

import jax
import jax.numpy as jnp
from jax import lax
from jax.experimental import pallas as pl
from jax.experimental.pallas import tpu as pltpu
from jax.experimental.pallas import tpu_sc as plsc

_N = 2048
_D = 768
_F = 16384
_KTOT = 32 * _N
_L1C = 1e-4

_EB = 1024
_DB = 512
_NB = 8
_RR = 11

_BND = tuple(((127 + e) << 23) | (h << 22)
             for e in range(-2, 2) for h in (0, 1))

_NWK = 32
_ROWS_W = _N // _NWK
_CCAP = 32768
_PLC = _CCAP // 16


def _prep_body(x_ref, xn_ref, mu_ref, sd_ref):
    x = x_ref[...]
    mu = jnp.mean(x, axis=1, keepdims=True)
    xc = x - mu
    m2 = jnp.mean(xc, axis=1, keepdims=True)
    var = jnp.sum((xc - m2) * (xc - m2), axis=1, keepdims=True) / (_D - 1)
    sd = jnp.sqrt(var)
    xn_ref[...] = xc / (sd + 1e-5)
    mu_ref[...] = jnp.broadcast_to(mu, (_N, 128))
    sd_ref[...] = jnp.broadcast_to(sd, (_N, 128))


def _enc_body(xn_ref, w_ref, b_ref, acts_ref, cnt_ref, cnt_sm):
    t = pl.program_id(0)
    nt = pl.num_programs(0)

    @pl.when(t == 0)
    def _():
        for i in range(len(_BND)):
            cnt_sm[i] = 0

    xc = xn_ref[...] - b_ref[...]
    acts = jax.lax.dot_general(xc, w_ref[...], (((1,), (0,)), ((), ())),
                               preferred_element_type=jnp.float32)
    acts = jnp.maximum(acts, 0.0)
    acts_ref[...] = acts
    bits = jax.lax.bitcast_convert_type(acts, jnp.int32)
    for i, b in enumerate(_BND):
        cnt_sm[i] = cnt_sm[i] + jnp.sum((bits >= b).astype(jnp.int32))

    @pl.when(t == nt - 1)
    def _():
        cnt_ref[...] = jnp.concatenate(
            [jnp.full((1, 128), cnt_sm[i], jnp.int32)
             for i in range(len(_BND))], axis=0)


def _compact_body(acts_hbm, lo_hbm, cand_hbm, cnt_hbm,
                  win_a, win_b, cand_v, lo_v, cnt_v, sem_a, sem_b):
    c = lax.axis_index("c")
    s = lax.axis_index("s")
    wid = s * 2 + c
    row0 = wid * _ROWS_W
    pltpu.sync_copy(lo_hbm, lo_v)
    lo = lo_v[...]

    def zb(i, _):
        cand_v[pl.ds(i * 16, 16)] = jnp.zeros((16,), jnp.float32)
        return 0

    lax.fori_loop(0, (_CCAP + 64) // 16, zb, 0)

    pltpu.async_copy(acts_hbm.at[row0], win_a, sem_a)

    iota16 = lax.iota(jnp.int32, 16)
    one16 = jnp.ones((16,), jnp.int32)
    zero16 = jnp.zeros((16,), jnp.int32)
    _UNR = 4
    _PLU = _CCAP // (16 * _UNR)

    def process(buf, cnts):
        def vb(i, cnts):
            base = i * (16 * _UNR)
            out = []
            for u in range(_UNR):
                v = buf[pl.ds(base + 16 * u, 16)]
                m = v >= lo
                idx = (jnp.minimum(cnts[u], _PLU - 1) * (16 * _UNR)
                       + (u * 16) + iota16)
                plsc.store_scatter(cand_v, [idx], v, mask=m)
                out.append(cnts[u] + jnp.where(m, one16, zero16))
            return tuple(out)
        return lax.fori_loop(0, _F // (16 * _UNR), vb, cnts)

    def gbody(g, cnts):
        r0 = row0 + 2 * g
        pltpu.async_copy(acts_hbm.at[r0 + 1], win_b, sem_b)
        pltpu.make_async_copy(acts_hbm.at[r0], win_a, sem_a).wait()
        cnts = process(win_a, cnts)

        @pl.when(g < _ROWS_W // 2 - 1)
        def _():
            pltpu.async_copy(acts_hbm.at[r0 + 2], win_a, sem_a)

        pltpu.make_async_copy(acts_hbm.at[r0 + 1], win_b, sem_b).wait()
        cnts = process(win_b, cnts)
        return cnts

    cnts0 = tuple(jnp.zeros((16,), jnp.int32) for _ in range(_UNR))
    cnts = lax.fori_loop(0, _ROWS_W // 2, gbody, cnts0)
    tot = zero16
    for u in range(_UNR):
        tot = tot + jnp.minimum(cnts[u], _PLU)
    cnt_v[...] = tot
    pltpu.sync_copy(cnt_v, cnt_hbm.at[wid])
    pltpu.sync_copy(cand_v.at[pl.ds(0, _CCAP)], cand_hbm.at[wid])


def _refine_body(lob_ref, cand_ref, tau_ref):
    def rbody(r, lh):
        lo, hi = lh
        step = jnp.maximum(jax.lax.div(hi - lo, _NB + 1), 1)
        bits = jax.lax.bitcast_convert_type(cand_ref[...], jnp.int32)
        nlo, nhi = lo, hi
        for i in range(_NB):
            b = lo + step * (i + 1)
            cval = jnp.sum((bits >= b).astype(jnp.int32))
            ok = cval >= _KTOT
            nlo = jnp.where(ok, jnp.maximum(nlo, b), nlo)
            nhi = jnp.where(ok, nhi, jnp.minimum(nhi, b))
        return (nlo, nhi)

    lo, hi = lax.fori_loop(0, _RR, rbody, (lob_ref[0], lob_ref[1]))
    tau_ref[...] = jnp.full((8, 128), lo, jnp.int32)


def _dec_body(tau_ref, acts_ref, w_ref, xn_ref, mu_ref, sd_ref, b_ref,
              atk_ref, sae_ref, l1p_ref, l0p_ref, l2p_ref, acc):
    t = pl.program_id(0)
    nt = pl.num_programs(0)
    tau = tau_ref[0]
    a = acts_ref[...]
    m = a >= tau
    atk = jnp.where(m, a, 0.0)
    atk_ref[...] = atk
    l1p_ref[...] = jnp.full((1, 1, 128), jnp.sum(atk), jnp.float32)
    l0p_ref[...] = jnp.full((1, 1, 128), jnp.sum(m.astype(jnp.float32)),
                            jnp.float32)

    @pl.when(t == 0)
    def _():
        acc[...] = jnp.zeros_like(acc)

    acc[...] += jax.lax.dot_general(atk, w_ref[...], (((1,), (0,)), ((), ())),
                                    preferred_element_type=jnp.float32)

    @pl.when(t == nt - 1)
    def _():
        xr = acc[...] + b_ref[...]
        xn = xn_ref[...]
        d = xr - xn
        l2p_ref[...] = jnp.full((1, 128), jnp.sum(d * d), jnp.float32)
        sae_ref[...] = xr * sd_ref[...][:, 0:1] + mu_ref[...][:, 0:1]


@jax.jit
def kernel(x, W_enc, W_dec, b_dec):
    f32 = jnp.float32

    xn, mu, sd = pl.pallas_call(
        _prep_body,
        out_shape=[
            jax.ShapeDtypeStruct((_N, _D), f32),
            jax.ShapeDtypeStruct((_N, 128), f32),
            jax.ShapeDtypeStruct((_N, 128), f32),
        ],
    )(x)

    n_eb = _F // _EB
    acts, cntlad = pl.pallas_call(
        _enc_body,
        grid=(n_eb,),
        in_specs=[
            pl.BlockSpec((_N, _D), lambda i: (0, 0)),
            pl.BlockSpec((_D, _EB), lambda i: (0, i)),
            pl.BlockSpec((_D,), lambda i: (0,)),
        ],
        out_specs=[
            pl.BlockSpec((_N, _EB), lambda i: (0, i)),
            pl.BlockSpec((len(_BND), 128), lambda i: (0, 0)),
        ],
        out_shape=[
            jax.ShapeDtypeStruct((_N, _F), f32),
            jax.ShapeDtypeStruct((len(_BND), 128), jnp.int32),
        ],
        scratch_shapes=[pltpu.SMEM((len(_BND),), jnp.int32)],
    )(xn, W_enc, b_dec)

    cnts = cntlad[:, 0]
    bnd_arr = jnp.array(_BND, jnp.int32)
    nok = jnp.sum((cnts >= _KTOT).astype(jnp.int32))
    lo_bits = jnp.where(nok > 0, bnd_arr[jnp.maximum(nok - 1, 0)],
                        jnp.int32(1))
    hi_bits = jnp.where(nok < len(_BND), bnd_arr[jnp.minimum(nok, 7)],
                        jnp.int32(0x7F800000))
    lo_vec = jnp.full((16,), lax.bitcast_convert_type(lo_bits, f32), f32)

    mesh = plsc.VectorSubcoreMesh(core_axis_name="c", subcore_axis_name="s")
    compact = functools.partial(
        pl.kernel,
        mesh=mesh,
        compiler_params=pltpu.CompilerParams(needs_layout_passes=False),
        out_type=[
            jax.ShapeDtypeStruct((_NWK, _CCAP), f32),
            jax.ShapeDtypeStruct((_NWK, 16), jnp.int32),
        ],
        scratch_types=[
            pltpu.VMEM((_F,), f32),
            pltpu.VMEM((_F,), f32),
            pltpu.VMEM((_CCAP + 64,), f32),
            pltpu.VMEM((16,), f32),
            pltpu.VMEM((16,), jnp.int32),
            pltpu.SemaphoreType.DMA,
            pltpu.SemaphoreType.DMA,
        ],
    )(_compact_body)
    cand, scnt = compact(acts, lo_vec)

    lob = jnp.stack([lo_bits, hi_bits])
    taub = pl.pallas_call(
        _refine_body,
        in_specs=[
            pl.BlockSpec(memory_space=pltpu.SMEM),
            pl.BlockSpec((_NWK, _CCAP), lambda: (0, 0)),
        ],
        out_specs=pl.BlockSpec((8, 128), lambda: (0, 0)),
        out_shape=jax.ShapeDtypeStruct((8, 128), jnp.int32),
    )(lob, cand)

    c_total = jnp.sum(scnt)
    tau = jnp.where(c_total >= _KTOT,
                    lax.bitcast_convert_type(taub[0, 0], f32),
                    jnp.float32(0.0))

    n_db = _F // _DB
    atk, sae, l1p, l0p, l2p = pl.pallas_call(
        _dec_body,
        grid=(n_db,),
        in_specs=[
            pl.BlockSpec(memory_space=pltpu.SMEM),
            pl.BlockSpec((_N, _DB), lambda i: (0, i)),
            pl.BlockSpec((_DB, _D), lambda i: (i, 0)),
            pl.BlockSpec((_N, _D), lambda i: (0, 0)),
            pl.BlockSpec((_N, 128), lambda i: (0, 0)),
            pl.BlockSpec((_N, 128), lambda i: (0, 0)),
            pl.BlockSpec((_D,), lambda i: (0,)),
        ],
        out_specs=[
            pl.BlockSpec((_N, _DB), lambda i: (0, i)),
            pl.BlockSpec((_N, _D), lambda i: (0, 0)),
            pl.BlockSpec((1, 1, 128), lambda i: (i, 0, 0)),
            pl.BlockSpec((1, 1, 128), lambda i: (i, 0, 0)),
            pl.BlockSpec((1, 128), lambda i: (0, 0)),
        ],
        out_shape=[
            jax.ShapeDtypeStruct((_N, _F), f32),
            jax.ShapeDtypeStruct((_N, _D), f32),
            jax.ShapeDtypeStruct((n_db, 1, 128), f32),
            jax.ShapeDtypeStruct((n_db, 1, 128), f32),
            jax.ShapeDtypeStruct((1, 128), f32),
        ],
        scratch_shapes=[pltpu.VMEM((_N, _D), f32)],
    )(jnp.reshape(tau, (1,)), acts, W_dec, xn, mu, sd, b_dec)

    l1n = jnp.sum(l1p[:, 0, 0]) / _N
    l0n = jnp.sum(l0p[:, 0, 0]) / _N
    l2 = l2p[0, 0] / (_N * _D)
    loss = l2 + jnp.float32(0.0)
    return sae, atk, loss, l2, _L1C * l1n, l0n, l1n


# speedup vs baseline: 59.5317x; 1.0151x over previous
import functools

import jax
import jax.numpy as jnp
from jax import lax
from jax.experimental import pallas as pl
from jax.experimental.pallas import tpu as pltpu
from jax.experimental.pallas import tpu_sc as plsc

_N = 2048
_D = 768
_F = 16384
_KTOT = 32 * _N
_L1C = 1e-4

_EB = 1024
_DB = 512
_NB = 8
_RR = 11

_BND = tuple(((127 + e) << 23) | (h << 22)
             for e in range(-2, 2) for h in (0, 1))

_NWK = 32
_ROWS_W = _N // _NWK
_CCAP = 32768
_SLAB = 8
_CW = 4096


def _prep_body(x_ref, xn_ref, mu_ref, sd_ref):
    x = x_ref[...]
    mu = jnp.mean(x, axis=1, keepdims=True)
    xc = x - mu
    m2 = jnp.mean(xc, axis=1, keepdims=True)
    var = jnp.sum((xc - m2) * (xc - m2), axis=1, keepdims=True) / (_D - 1)
    sd = jnp.sqrt(var)
    xn_ref[...] = xc / (sd + 1e-5)
    mu_ref[...] = jnp.broadcast_to(mu, (_N, 128))
    sd_ref[...] = jnp.broadcast_to(sd, (_N, 128))


def _enc_body(xn_ref, w_ref, b_ref, acts_ref, cnt_ref, cnt_sm):
    t = pl.program_id(0)
    nt = pl.num_programs(0)

    @pl.when(t == 0)
    def _():
        for i in range(len(_BND)):
            cnt_sm[i] = 0

    xc = xn_ref[...] - b_ref[...]
    acts = jax.lax.dot_general(xc, w_ref[...], (((1,), (0,)), ((), ())),
                               preferred_element_type=jnp.float32)
    acts = jnp.maximum(acts, 0.0)
    acts_ref[...] = acts
    bits = jax.lax.bitcast_convert_type(acts, jnp.int32)
    for i, b in enumerate(_BND):
        cnt_sm[i] = cnt_sm[i] + jnp.sum((bits >= b).astype(jnp.int32))

    @pl.when(t == nt - 1)
    def _():
        cnt_ref[...] = jnp.concatenate(
            [jnp.full((1, 128), cnt_sm[i], jnp.int32)
             for i in range(len(_BND))], axis=0)


def _compact_body(acts_hbm, lo_hbm, cand_hbm, cnt_hbm,
                  win_a, win_b, cand_v, lo_v, cnt_v, sem_a, sem_b):
    c = lax.axis_index("c")
    s = lax.axis_index("s")
    wid = s * 2 + c
    row0 = wid * _ROWS_W
    pltpu.sync_copy(lo_hbm, lo_v)
    lo = lo_v[...]

    def zb(i, _):
        cand_v[pl.ds(i * 16, 16)] = jnp.zeros((16,), jnp.float32)
        return 0

    lax.fori_loop(0, (_CCAP + 64) // 16, zb, 0)

    def src(w):
        s = jax.lax.div(w, 4)
        q = jax.lax.rem(w, 4)
        return acts_hbm.at[pl.ds(row0 + _SLAB * s, _SLAB),
                           pl.ds(_CW * q, _CW)]

    pltpu.async_copy(src(0), win_a, sem_a)

    iota16 = lax.iota(jnp.int32, 16)
    one16 = jnp.ones((16,), jnp.int32)
    zero16 = jnp.zeros((16,), jnp.int32)
    _UNR = 4
    _PLU = _CCAP // (16 * _UNR)

    def process(buf, cnts):
        def vb(i, cnts):
            base = i * (16 * _UNR)
            out = list(cnts)
            for r in range(_SLAB):
                for u in range(_UNR):
                    v = buf[r, pl.ds(base + 16 * u, 16)]
                    m = v >= lo
                    idx = (jnp.minimum(out[u], _PLU - 1) * (16 * _UNR)
                           + (u * 16) + iota16)
                    plsc.store_scatter(cand_v, [idx], v, mask=m)
                    out[u] = out[u] + jnp.where(m, one16, zero16)
            return tuple(out)
        return lax.fori_loop(0, _CW // (16 * _UNR), vb, cnts)

    nwin = (_ROWS_W // _SLAB) * (_F // _CW)

    def gbody(g, cnts):
        w0 = 2 * g
        pltpu.async_copy(src(w0 + 1), win_b, sem_b)
        pltpu.make_async_copy(src(w0), win_a, sem_a).wait()
        cnts = process(win_a, cnts)

        @pl.when(g < nwin // 2 - 1)
        def _():
            pltpu.async_copy(src(w0 + 2), win_a, sem_a)

        pltpu.make_async_copy(src(w0 + 1), win_b, sem_b).wait()
        cnts = process(win_b, cnts)
        return cnts

    cnts0 = tuple(jnp.zeros((16,), jnp.int32) for _ in range(_UNR))
    cnts = lax.fori_loop(0, nwin // 2, gbody, cnts0)
    tot = zero16
    for u in range(_UNR):
        tot = tot + jnp.minimum(cnts[u], _PLU)
    cnt_v[...] = tot
    pltpu.sync_copy(cnt_v, cnt_hbm.at[wid])
    pltpu.sync_copy(cand_v.at[pl.ds(0, _CCAP)], cand_hbm.at[wid])


def _refine_body(lob_ref, cand_ref, tau_ref):
    def rbody(r, lh):
        lo, hi = lh
        step = jnp.maximum(jax.lax.div(hi - lo, _NB + 1), 1)
        bits = jax.lax.bitcast_convert_type(cand_ref[...], jnp.int32)
        nlo, nhi = lo, hi
        for i in range(_NB):
            b = lo + step * (i + 1)
            cval = jnp.sum((bits >= b).astype(jnp.int32))
            ok = cval >= _KTOT
            nlo = jnp.where(ok, jnp.maximum(nlo, b), nlo)
            nhi = jnp.where(ok, nhi, jnp.minimum(nhi, b))
        return (nlo, nhi)

    lo, hi = lax.fori_loop(0, _RR, rbody, (lob_ref[0], lob_ref[1]))
    tau_ref[...] = jnp.full((8, 128), lo, jnp.int32)


def _dec_body(tau_ref, acts_ref, w_ref, xn_ref, mu_ref, sd_ref, b_ref,
              atk_ref, sae_ref, l1p_ref, l0p_ref, l2p_ref, acc):
    t = pl.program_id(0)
    nt = pl.num_programs(0)
    tau = tau_ref[0]
    a = acts_ref[...]
    m = a >= tau
    atk = jnp.where(m, a, 0.0)
    atk_ref[...] = atk
    l1p_ref[...] = jnp.full((1, 1, 128), jnp.sum(atk), jnp.float32)
    l0p_ref[...] = jnp.full((1, 1, 128), jnp.sum(m.astype(jnp.float32)),
                            jnp.float32)

    @pl.when(t == 0)
    def _():
        acc[...] = jnp.zeros_like(acc)

    acc[...] += jax.lax.dot_general(atk, w_ref[...], (((1,), (0,)), ((), ())),
                                    preferred_element_type=jnp.float32)

    @pl.when(t == nt - 1)
    def _():
        xr = acc[...] + b_ref[...]
        xn = xn_ref[...]
        d = xr - xn
        l2p_ref[...] = jnp.full((1, 128), jnp.sum(d * d), jnp.float32)
        sae_ref[...] = xr * sd_ref[...][:, 0:1] + mu_ref[...][:, 0:1]


@jax.jit
def kernel(x, W_enc, W_dec, b_dec):
    f32 = jnp.float32

    xn, mu, sd = pl.pallas_call(
        _prep_body,
        out_shape=[
            jax.ShapeDtypeStruct((_N, _D), f32),
            jax.ShapeDtypeStruct((_N, 128), f32),
            jax.ShapeDtypeStruct((_N, 128), f32),
        ],
    )(x)

    n_eb = _F // _EB
    acts, cntlad = pl.pallas_call(
        _enc_body,
        grid=(n_eb,),
        in_specs=[
            pl.BlockSpec((_N, _D), lambda i: (0, 0)),
            pl.BlockSpec((_D, _EB), lambda i: (0, i)),
            pl.BlockSpec((_D,), lambda i: (0,)),
        ],
        out_specs=[
            pl.BlockSpec((_N, _EB), lambda i: (0, i)),
            pl.BlockSpec((len(_BND), 128), lambda i: (0, 0)),
        ],
        out_shape=[
            jax.ShapeDtypeStruct((_N, _F), f32),
            jax.ShapeDtypeStruct((len(_BND), 128), jnp.int32),
        ],
        scratch_shapes=[pltpu.SMEM((len(_BND),), jnp.int32)],
    )(xn, W_enc, b_dec)

    cnts = cntlad[:, 0]
    bnd_arr = jnp.array(_BND, jnp.int32)
    nok = jnp.sum((cnts >= _KTOT).astype(jnp.int32))
    lo_bits = jnp.where(nok > 0, bnd_arr[jnp.maximum(nok - 1, 0)],
                        jnp.int32(1))
    hi_bits = jnp.where(nok < len(_BND), bnd_arr[jnp.minimum(nok, 7)],
                        jnp.int32(0x7F800000))
    lo_vec = jnp.full((16,), lax.bitcast_convert_type(lo_bits, f32), f32)

    mesh = plsc.VectorSubcoreMesh(core_axis_name="c", subcore_axis_name="s")
    compact = functools.partial(
        pl.kernel,
        mesh=mesh,
        compiler_params=pltpu.CompilerParams(needs_layout_passes=False),
        out_type=[
            jax.ShapeDtypeStruct((_NWK, _CCAP), f32),
            jax.ShapeDtypeStruct((_NWK, 16), jnp.int32),
        ],
        scratch_types=[
            pltpu.VMEM((_SLAB, _CW), f32),
            pltpu.VMEM((_SLAB, _CW), f32),
            pltpu.VMEM((_CCAP + 64,), f32),
            pltpu.VMEM((16,), f32),
            pltpu.VMEM((16,), jnp.int32),
            pltpu.SemaphoreType.DMA,
            pltpu.SemaphoreType.DMA,
        ],
    )(_compact_body)
    cand, scnt = compact(acts, lo_vec)

    lob = jnp.stack([lo_bits, hi_bits])
    taub = pl.pallas_call(
        _refine_body,
        in_specs=[
            pl.BlockSpec(memory_space=pltpu.SMEM),
            pl.BlockSpec((_NWK, _CCAP), lambda: (0, 0)),
        ],
        out_specs=pl.BlockSpec((8, 128), lambda: (0, 0)),
        out_shape=jax.ShapeDtypeStruct((8, 128), jnp.int32),
    )(lob, cand)

    c_total = jnp.sum(scnt)
    tau = jnp.where(c_total >= _KTOT,
                    lax.bitcast_convert_type(taub[0, 0], f32),
                    jnp.float32(0.0))

    n_db = _F // _DB
    atk, sae, l1p, l0p, l2p = pl.pallas_call(
        _dec_body,
        grid=(n_db,),
        in_specs=[
            pl.BlockSpec(memory_space=pltpu.SMEM),
            pl.BlockSpec((_N, _DB), lambda i: (0, i)),
            pl.BlockSpec((_DB, _D), lambda i: (i, 0)),
            pl.BlockSpec((_N, _D), lambda i: (0, 0)),
            pl.BlockSpec((_N, 128), lambda i: (0, 0)),
            pl.BlockSpec((_N, 128), lambda i: (0, 0)),
            pl.BlockSpec((_D,), lambda i: (0,)),
        ],
        out_specs=[
            pl.BlockSpec((_N, _DB), lambda i: (0, i)),
            pl.BlockSpec((_N, _D), lambda i: (0, 0)),
            pl.BlockSpec((1, 1, 128), lambda i: (i, 0, 0)),
            pl.BlockSpec((1, 1, 128), lambda i: (i, 0, 0)),
            pl.BlockSpec((1, 128), lambda i: (0, 0)),
        ],
        out_shape=[
            jax.ShapeDtypeStruct((_N, _F), f32),
            jax.ShapeDtypeStruct((_N, _D), f32),
            jax.ShapeDtypeStruct((n_db, 1, 128), f32),
            jax.ShapeDtypeStruct((n_db, 1, 128), f32),
            jax.ShapeDtypeStruct((1, 128), f32),
        ],
        scratch_shapes=[pltpu.VMEM((_N, _D), f32)],
    )(jnp.reshape(tau, (1,)), acts, W_dec, xn, mu, sd, b_dec)

    l1n = jnp.sum(l1p[:, 0, 0]) / _N
    l0n = jnp.sum(l0p[:, 0, 0]) / _N
    l2 = l2p[0, 0] / (_N * _D)
    loss = l2 + jnp.float32(0.0)
    return sae, atk, loss, l2, _L1C * l1n, l0n, l1n


# speedup vs baseline: 61.5142x; 1.0333x over previous
import functools

import jax
import jax.numpy as jnp
from jax import lax
from jax.experimental import pallas as pl
from jax.experimental.pallas import tpu as pltpu
from jax.experimental.pallas import tpu_sc as plsc

_N = 2048
_D = 768
_F = 16384
_KTOT = 32 * _N
_L1C = 1e-4

_EB = 1024
_DB = 512
_NB = 8
_RR = 11

_BND = tuple(((127 + e) << 23) | (h << 22)
             for e in range(-2, 2) for h in (0, 1))

_NWK = 32
_ROWS_W = _N // _NWK
_CCAP = 32768
_SLAB = 8
_CW = 4096


def _prep_body(x_ref, xn_ref, mu_ref, sd_ref):
    x = x_ref[...]
    mu = jnp.mean(x, axis=1, keepdims=True)
    xc = x - mu
    m2 = jnp.mean(xc, axis=1, keepdims=True)
    var = jnp.sum((xc - m2) * (xc - m2), axis=1, keepdims=True) / (_D - 1)
    sd = jnp.sqrt(var)
    xn_ref[...] = xc / (sd + 1e-5)
    mu_ref[...] = jnp.broadcast_to(mu, (_N, 128))
    sd_ref[...] = jnp.broadcast_to(sd, (_N, 128))


def _enc_body(xn_ref, w_ref, b_ref, acts_ref, cnt_ref, cnt_sm):
    t = pl.program_id(0)
    nt = pl.num_programs(0)

    @pl.when(t == 0)
    def _():
        for i in range(len(_BND)):
            cnt_sm[i] = 0

    xc = xn_ref[...] - b_ref[...]
    acts = jax.lax.dot_general(xc, w_ref[...], (((1,), (0,)), ((), ())),
                               preferred_element_type=jnp.float32)
    acts = jnp.maximum(acts, 0.0)
    acts_ref[...] = acts
    bits = jax.lax.bitcast_convert_type(acts, jnp.int32)
    for i, b in enumerate(_BND):
        cnt_sm[i] = cnt_sm[i] + jnp.sum((bits >= b).astype(jnp.int32))

    @pl.when(t == nt - 1)
    def _():
        cnt_ref[...] = jnp.concatenate(
            [jnp.full((1, 128), cnt_sm[i], jnp.int32)
             for i in range(len(_BND))], axis=0)


def _compact_body(acts_hbm, lo_hbm, cand_hbm, cnt_hbm,
                  win_a, win_b, cand_v, lo_v, cnt_v, sem_a, sem_b):
    c = lax.axis_index("c")
    s = lax.axis_index("s")
    wid = s * 2 + c
    row0 = wid * _ROWS_W
    pltpu.sync_copy(lo_hbm, lo_v)
    lo = lo_v[...]

    def zb(i, _):
        cand_v[pl.ds(i * 16, 16)] = jnp.zeros((16,), jnp.float32)
        return 0

    lax.fori_loop(0, (_CCAP + 64) // 16, zb, 0)

    def src(w):
        s = jax.lax.div(w, 4)
        q = jax.lax.rem(w, 4)
        return acts_hbm.at[pl.ds(row0 + _SLAB * s, _SLAB),
                           pl.ds(_CW * q, _CW)]

    pltpu.async_copy(src(0), win_a, sem_a)

    iota16 = lax.iota(jnp.int32, 16)
    one16 = jnp.ones((16,), jnp.int32)
    zero16 = jnp.zeros((16,), jnp.int32)
    _UNR = 4
    _PLU = _CCAP // (16 * _UNR)

    def process(buf, cnts):
        def vb(i, cnts):
            base = i * (16 * _UNR)
            out = list(cnts)
            for r in range(_SLAB):
                for u in range(_UNR):
                    v = buf[r, pl.ds(base + 16 * u, 16)]
                    m = v >= lo
                    idx = (jnp.minimum(out[u], _PLU - 1) * (16 * _UNR)
                           + (u * 16) + iota16)
                    plsc.store_scatter(cand_v, [idx], v, mask=m)
                    out[u] = out[u] + jnp.where(m, one16, zero16)
            return tuple(out)
        return plsc.parallel_loop(0, _CW // (16 * _UNR), unroll=2,
                                  carry=tuple(cnts))(vb)

    nwin = (_ROWS_W // _SLAB) * (_F // _CW)

    def gbody(g, cnts):
        w0 = 2 * g
        pltpu.async_copy(src(w0 + 1), win_b, sem_b)
        pltpu.make_async_copy(src(w0), win_a, sem_a).wait()
        cnts = process(win_a, cnts)

        @pl.when(g < nwin // 2 - 1)
        def _():
            pltpu.async_copy(src(w0 + 2), win_a, sem_a)

        pltpu.make_async_copy(src(w0 + 1), win_b, sem_b).wait()
        cnts = process(win_b, cnts)
        return cnts

    cnts0 = tuple(jnp.zeros((16,), jnp.int32) for _ in range(_UNR))
    cnts = lax.fori_loop(0, nwin // 2, gbody, cnts0)
    tot = zero16
    for u in range(_UNR):
        tot = tot + jnp.minimum(cnts[u], _PLU)
    cnt_v[...] = tot
    pltpu.sync_copy(cnt_v, cnt_hbm.at[wid])
    pltpu.sync_copy(cand_v.at[pl.ds(0, _CCAP)], cand_hbm.at[wid])


def _refine_body(lob_ref, cand_ref, tau_ref):
    def rbody(r, lh):
        lo, hi = lh
        step = jnp.maximum(jax.lax.div(hi - lo, _NB + 1), 1)
        bits = jax.lax.bitcast_convert_type(cand_ref[...], jnp.int32)
        nlo, nhi = lo, hi
        for i in range(_NB):
            b = lo + step * (i + 1)
            cval = jnp.sum((bits >= b).astype(jnp.int32))
            ok = cval >= _KTOT
            nlo = jnp.where(ok, jnp.maximum(nlo, b), nlo)
            nhi = jnp.where(ok, nhi, jnp.minimum(nhi, b))
        return (nlo, nhi)

    lo, hi = lax.fori_loop(0, _RR, rbody, (lob_ref[0], lob_ref[1]))
    tau_ref[...] = jnp.full((8, 128), lo, jnp.int32)


def _dec_body(tau_ref, acts_ref, w_ref, xn_ref, mu_ref, sd_ref, b_ref,
              atk_ref, sae_ref, l1p_ref, l0p_ref, l2p_ref, acc):
    t = pl.program_id(0)
    nt = pl.num_programs(0)
    tau = tau_ref[0]
    a = acts_ref[...]
    m = a >= tau
    atk = jnp.where(m, a, 0.0)
    atk_ref[...] = atk
    l1p_ref[...] = jnp.full((1, 1, 128), jnp.sum(atk), jnp.float32)
    l0p_ref[...] = jnp.full((1, 1, 128), jnp.sum(m.astype(jnp.float32)),
                            jnp.float32)

    @pl.when(t == 0)
    def _():
        acc[...] = jnp.zeros_like(acc)

    acc[...] += jax.lax.dot_general(atk, w_ref[...], (((1,), (0,)), ((), ())),
                                    preferred_element_type=jnp.float32)

    @pl.when(t == nt - 1)
    def _():
        xr = acc[...] + b_ref[...]
        xn = xn_ref[...]
        d = xr - xn
        l2p_ref[...] = jnp.full((1, 128), jnp.sum(d * d), jnp.float32)
        sae_ref[...] = xr * sd_ref[...][:, 0:1] + mu_ref[...][:, 0:1]


@jax.jit
def kernel(x, W_enc, W_dec, b_dec):
    f32 = jnp.float32

    xn, mu, sd = pl.pallas_call(
        _prep_body,
        out_shape=[
            jax.ShapeDtypeStruct((_N, _D), f32),
            jax.ShapeDtypeStruct((_N, 128), f32),
            jax.ShapeDtypeStruct((_N, 128), f32),
        ],
    )(x)

    n_eb = _F // _EB
    acts, cntlad = pl.pallas_call(
        _enc_body,
        grid=(n_eb,),
        in_specs=[
            pl.BlockSpec((_N, _D), lambda i: (0, 0)),
            pl.BlockSpec((_D, _EB), lambda i: (0, i)),
            pl.BlockSpec((_D,), lambda i: (0,)),
        ],
        out_specs=[
            pl.BlockSpec((_N, _EB), lambda i: (0, i)),
            pl.BlockSpec((len(_BND), 128), lambda i: (0, 0)),
        ],
        out_shape=[
            jax.ShapeDtypeStruct((_N, _F), f32),
            jax.ShapeDtypeStruct((len(_BND), 128), jnp.int32),
        ],
        scratch_shapes=[pltpu.SMEM((len(_BND),), jnp.int32)],
    )(xn, W_enc, b_dec)

    cnts = cntlad[:, 0]
    bnd_arr = jnp.array(_BND, jnp.int32)
    nok = jnp.sum((cnts >= _KTOT).astype(jnp.int32))
    lo_bits = jnp.where(nok > 0, bnd_arr[jnp.maximum(nok - 1, 0)],
                        jnp.int32(1))
    hi_bits = jnp.where(nok < len(_BND), bnd_arr[jnp.minimum(nok, 7)],
                        jnp.int32(0x7F800000))
    lo_vec = jnp.full((16,), lax.bitcast_convert_type(lo_bits, f32), f32)

    mesh = plsc.VectorSubcoreMesh(core_axis_name="c", subcore_axis_name="s")
    compact = functools.partial(
        pl.kernel,
        mesh=mesh,
        compiler_params=pltpu.CompilerParams(needs_layout_passes=False),
        out_type=[
            jax.ShapeDtypeStruct((_NWK, _CCAP), f32),
            jax.ShapeDtypeStruct((_NWK, 16), jnp.int32),
        ],
        scratch_types=[
            pltpu.VMEM((_SLAB, _CW), f32),
            pltpu.VMEM((_SLAB, _CW), f32),
            pltpu.VMEM((_CCAP + 64,), f32),
            pltpu.VMEM((16,), f32),
            pltpu.VMEM((16,), jnp.int32),
            pltpu.SemaphoreType.DMA,
            pltpu.SemaphoreType.DMA,
        ],
    )(_compact_body)
    cand, scnt = compact(acts, lo_vec)

    lob = jnp.stack([lo_bits, hi_bits])
    taub = pl.pallas_call(
        _refine_body,
        in_specs=[
            pl.BlockSpec(memory_space=pltpu.SMEM),
            pl.BlockSpec((_NWK, _CCAP), lambda: (0, 0)),
        ],
        out_specs=pl.BlockSpec((8, 128), lambda: (0, 0)),
        out_shape=jax.ShapeDtypeStruct((8, 128), jnp.int32),
    )(lob, cand)

    c_total = jnp.sum(scnt)
    tau = jnp.where(c_total >= _KTOT,
                    lax.bitcast_convert_type(taub[0, 0], f32),
                    jnp.float32(0.0))

    n_db = _F // _DB
    atk, sae, l1p, l0p, l2p = pl.pallas_call(
        _dec_body,
        grid=(n_db,),
        in_specs=[
            pl.BlockSpec(memory_space=pltpu.SMEM),
            pl.BlockSpec((_N, _DB), lambda i: (0, i)),
            pl.BlockSpec((_DB, _D), lambda i: (i, 0)),
            pl.BlockSpec((_N, _D), lambda i: (0, 0)),
            pl.BlockSpec((_N, 128), lambda i: (0, 0)),
            pl.BlockSpec((_N, 128), lambda i: (0, 0)),
            pl.BlockSpec((_D,), lambda i: (0,)),
        ],
        out_specs=[
            pl.BlockSpec((_N, _DB), lambda i: (0, i)),
            pl.BlockSpec((_N, _D), lambda i: (0, 0)),
            pl.BlockSpec((1, 1, 128), lambda i: (i, 0, 0)),
            pl.BlockSpec((1, 1, 128), lambda i: (i, 0, 0)),
            pl.BlockSpec((1, 128), lambda i: (0, 0)),
        ],
        out_shape=[
            jax.ShapeDtypeStruct((_N, _F), f32),
            jax.ShapeDtypeStruct((_N, _D), f32),
            jax.ShapeDtypeStruct((n_db, 1, 128), f32),
            jax.ShapeDtypeStruct((n_db, 1, 128), f32),
            jax.ShapeDtypeStruct((1, 128), f32),
        ],
        scratch_shapes=[pltpu.VMEM((_N, _D), f32)],
    )(jnp.reshape(tau, (1,)), acts, W_dec, xn, mu, sd, b_dec)

    l1n = jnp.sum(l1p[:, 0, 0]) / _N
    l0n = jnp.sum(l0p[:, 0, 0]) / _N
    l2 = l2p[0, 0] / (_N * _D)
    loss = l2 + jnp.float32(0.0)
    return sae, atk, loss, l2, _L1C * l1n, l0n, l1n


# speedup vs baseline: 65.7570x; 1.0690x over previous
import functools

import jax
import jax.numpy as jnp
from jax import lax
from jax.experimental import pallas as pl
from jax.experimental.pallas import tpu as pltpu
from jax.experimental.pallas import tpu_sc as plsc

_N = 2048
_D = 768
_F = 16384
_KTOT = 32 * _N
_L1C = 1e-4

_EB = 1024
_DB = 512
_NB = 8
_RR = 11

_BND = ((126 << 23) | (1 << 22), 127 << 23,
        (127 << 23) | (1 << 22), 128 << 23)

_NWK = 32
_ROWS_W = _N // _NWK
_CCAP = 32768
_SLAB = 8
_CW = 4096


def _prep_body(x_ref, xn_ref, mu_ref, sd_ref):
    x = x_ref[...]
    mu = jnp.mean(x, axis=1, keepdims=True)
    xc = x - mu
    m2 = jnp.mean(xc, axis=1, keepdims=True)
    var = jnp.sum((xc - m2) * (xc - m2), axis=1, keepdims=True) / (_D - 1)
    sd = jnp.sqrt(var)
    xn_ref[...] = xc / (sd + 1e-5)
    mu_ref[...] = jnp.broadcast_to(mu, (_N, 128))
    sd_ref[...] = jnp.broadcast_to(sd, (_N, 128))


def _enc_body(xn_ref, w_ref, b_ref, acts_ref, cnt_ref, cnt_sm):
    t = pl.program_id(0)
    nt = pl.num_programs(0)

    @pl.when(t == 0)
    def _():
        for i in range(len(_BND)):
            cnt_sm[i] = 0

    xc = xn_ref[...] - b_ref[...]
    acts = jax.lax.dot_general(xc, w_ref[...], (((1,), (0,)), ((), ())),
                               preferred_element_type=jnp.float32)
    acts = jnp.maximum(acts, 0.0)
    acts_ref[...] = acts
    bits = jax.lax.bitcast_convert_type(acts, jnp.int32)
    for i, b in enumerate(_BND):
        cnt_sm[i] = cnt_sm[i] + jnp.sum((bits >= b).astype(jnp.int32))

    @pl.when(t == nt - 1)
    def _():
        cnt_ref[...] = jnp.concatenate(
            [jnp.full((1, 128), cnt_sm[i], jnp.int32)
             for i in range(len(_BND))], axis=0)


def _compact_body(acts_hbm, lo_hbm, cand_hbm, cnt_hbm,
                  win_a, win_b, cand_v, lo_v, cnt_v, sem_a, sem_b):
    c = lax.axis_index("c")
    s = lax.axis_index("s")
    wid = s * 2 + c
    row0 = wid * _ROWS_W
    pltpu.sync_copy(lo_hbm, lo_v)
    lo = lo_v[...]

    def zb(i, _):
        cand_v[pl.ds(i * 16, 16)] = jnp.zeros((16,), jnp.float32)
        return 0

    lax.fori_loop(0, (_CCAP + 64) // 16, zb, 0)

    def src(w):
        s = jax.lax.div(w, 4)
        q = jax.lax.rem(w, 4)
        return acts_hbm.at[pl.ds(row0 + _SLAB * s, _SLAB),
                           pl.ds(_CW * q, _CW)]

    pltpu.async_copy(src(0), win_a, sem_a)

    iota16 = lax.iota(jnp.int32, 16)
    one16 = jnp.ones((16,), jnp.int32)
    zero16 = jnp.zeros((16,), jnp.int32)
    _UNR = 4
    _PLU = _CCAP // (16 * _UNR)

    def process(buf, cnts):
        def vb(i, cnts):
            base = i * (16 * _UNR)
            out = list(cnts)
            for r in range(_SLAB):
                for u in range(_UNR):
                    v = buf[r, pl.ds(base + 16 * u, 16)]
                    m = v >= lo
                    idx = (jnp.minimum(out[u], _PLU - 1) * (16 * _UNR)
                           + (u * 16) + iota16)
                    plsc.store_scatter(cand_v, [idx], v, mask=m)
                    out[u] = out[u] + jnp.where(m, one16, zero16)
            return tuple(out)
        return plsc.parallel_loop(0, _CW // (16 * _UNR), unroll=2,
                                  carry=tuple(cnts))(vb)

    nwin = (_ROWS_W // _SLAB) * (_F // _CW)

    def gbody(g, cnts):
        w0 = 2 * g
        pltpu.async_copy(src(w0 + 1), win_b, sem_b)
        pltpu.make_async_copy(src(w0), win_a, sem_a).wait()
        cnts = process(win_a, cnts)

        @pl.when(g < nwin // 2 - 1)
        def _():
            pltpu.async_copy(src(w0 + 2), win_a, sem_a)

        pltpu.make_async_copy(src(w0 + 1), win_b, sem_b).wait()
        cnts = process(win_b, cnts)
        return cnts

    cnts0 = tuple(jnp.zeros((16,), jnp.int32) for _ in range(_UNR))
    cnts = lax.fori_loop(0, nwin // 2, gbody, cnts0)
    tot = zero16
    for u in range(_UNR):
        tot = tot + jnp.minimum(cnts[u], _PLU)
    cnt_v[...] = tot
    pltpu.sync_copy(cnt_v, cnt_hbm.at[wid])
    pltpu.sync_copy(cand_v.at[pl.ds(0, _CCAP)], cand_hbm.at[wid])


def _refine_body(lob_ref, cand_ref, tau_ref):
    def rbody(r, lh):
        lo, hi = lh
        step = jnp.maximum(jax.lax.div(hi - lo, _NB + 1), 1)
        bits = jax.lax.bitcast_convert_type(cand_ref[...], jnp.int32)
        nlo, nhi = lo, hi
        for i in range(_NB):
            b = lo + step * (i + 1)
            cval = jnp.sum((bits >= b).astype(jnp.int32))
            ok = cval >= _KTOT
            nlo = jnp.where(ok, jnp.maximum(nlo, b), nlo)
            nhi = jnp.where(ok, nhi, jnp.minimum(nhi, b))
        return (nlo, nhi)

    lo, hi = lax.fori_loop(0, _RR, rbody, (lob_ref[0], lob_ref[1]))
    tau_ref[...] = jnp.full((8, 128), lo, jnp.int32)


def _dec_body(tau_ref, acts_ref, w_ref, xn_ref, mu_ref, sd_ref, b_ref,
              atk_ref, sae_ref, l1p_ref, l0p_ref, l2p_ref, acc):
    t = pl.program_id(0)
    nt = pl.num_programs(0)
    tau = tau_ref[0]
    a = acts_ref[...]
    m = a >= tau
    atk = jnp.where(m, a, 0.0)
    atk_ref[...] = atk
    l1p_ref[...] = jnp.full((1, 1, 128), jnp.sum(atk), jnp.float32)
    l0p_ref[...] = jnp.full((1, 1, 128), jnp.sum(m.astype(jnp.float32)),
                            jnp.float32)

    @pl.when(t == 0)
    def _():
        acc[...] = jnp.zeros_like(acc)

    acc[...] += jax.lax.dot_general(atk, w_ref[...], (((1,), (0,)), ((), ())),
                                    preferred_element_type=jnp.float32)

    @pl.when(t == nt - 1)
    def _():
        xr = acc[...] + b_ref[...]
        xn = xn_ref[...]
        d = xr - xn
        l2p_ref[...] = jnp.full((1, 128), jnp.sum(d * d), jnp.float32)
        sae_ref[...] = xr * sd_ref[...][:, 0:1] + mu_ref[...][:, 0:1]


@jax.jit
def kernel(x, W_enc, W_dec, b_dec):
    f32 = jnp.float32

    xn, mu, sd = pl.pallas_call(
        _prep_body,
        out_shape=[
            jax.ShapeDtypeStruct((_N, _D), f32),
            jax.ShapeDtypeStruct((_N, 128), f32),
            jax.ShapeDtypeStruct((_N, 128), f32),
        ],
    )(x)

    n_eb = _F // _EB
    acts, cntlad = pl.pallas_call(
        _enc_body,
        grid=(n_eb,),
        in_specs=[
            pl.BlockSpec((_N, _D), lambda i: (0, 0)),
            pl.BlockSpec((_D, _EB), lambda i: (0, i)),
            pl.BlockSpec((_D,), lambda i: (0,)),
        ],
        out_specs=[
            pl.BlockSpec((_N, _EB), lambda i: (0, i)),
            pl.BlockSpec((len(_BND), 128), lambda i: (0, 0)),
        ],
        out_shape=[
            jax.ShapeDtypeStruct((_N, _F), f32),
            jax.ShapeDtypeStruct((len(_BND), 128), jnp.int32),
        ],
        scratch_shapes=[pltpu.SMEM((len(_BND),), jnp.int32)],
    )(xn, W_enc, b_dec)

    cnts = cntlad[:, 0]
    bnd_arr = jnp.array(_BND, jnp.int32)
    nok = jnp.sum((cnts >= _KTOT).astype(jnp.int32))
    lo_bits = jnp.where(nok > 0, bnd_arr[jnp.maximum(nok - 1, 0)],
                        jnp.int32(1))
    hi_bits = jnp.where(nok < len(_BND),
                        bnd_arr[jnp.minimum(nok, len(_BND) - 1)],
                        jnp.int32(0x7F800000))
    lo_vec = jnp.full((16,), lax.bitcast_convert_type(lo_bits, f32), f32)

    mesh = plsc.VectorSubcoreMesh(core_axis_name="c", subcore_axis_name="s")
    compact = functools.partial(
        pl.kernel,
        mesh=mesh,
        compiler_params=pltpu.CompilerParams(needs_layout_passes=False),
        out_type=[
            jax.ShapeDtypeStruct((_NWK, _CCAP), f32),
            jax.ShapeDtypeStruct((_NWK, 16), jnp.int32),
        ],
        scratch_types=[
            pltpu.VMEM((_SLAB, _CW), f32),
            pltpu.VMEM((_SLAB, _CW), f32),
            pltpu.VMEM((_CCAP + 64,), f32),
            pltpu.VMEM((16,), f32),
            pltpu.VMEM((16,), jnp.int32),
            pltpu.SemaphoreType.DMA,
            pltpu.SemaphoreType.DMA,
        ],
    )(_compact_body)
    cand, scnt = compact(acts, lo_vec)

    lob = jnp.stack([lo_bits, hi_bits])
    taub = pl.pallas_call(
        _refine_body,
        in_specs=[
            pl.BlockSpec(memory_space=pltpu.SMEM),
            pl.BlockSpec((_NWK, _CCAP), lambda: (0, 0)),
        ],
        out_specs=pl.BlockSpec((8, 128), lambda: (0, 0)),
        out_shape=jax.ShapeDtypeStruct((8, 128), jnp.int32),
    )(lob, cand)

    c_total = jnp.sum(scnt)
    tau = jnp.where(c_total >= _KTOT,
                    lax.bitcast_convert_type(taub[0, 0], f32),
                    jnp.float32(0.0))

    n_db = _F // _DB
    atk, sae, l1p, l0p, l2p = pl.pallas_call(
        _dec_body,
        grid=(n_db,),
        in_specs=[
            pl.BlockSpec(memory_space=pltpu.SMEM),
            pl.BlockSpec((_N, _DB), lambda i: (0, i)),
            pl.BlockSpec((_DB, _D), lambda i: (i, 0)),
            pl.BlockSpec((_N, _D), lambda i: (0, 0)),
            pl.BlockSpec((_N, 128), lambda i: (0, 0)),
            pl.BlockSpec((_N, 128), lambda i: (0, 0)),
            pl.BlockSpec((_D,), lambda i: (0,)),
        ],
        out_specs=[
            pl.BlockSpec((_N, _DB), lambda i: (0, i)),
            pl.BlockSpec((_N, _D), lambda i: (0, 0)),
            pl.BlockSpec((1, 1, 128), lambda i: (i, 0, 0)),
            pl.BlockSpec((1, 1, 128), lambda i: (i, 0, 0)),
            pl.BlockSpec((1, 128), lambda i: (0, 0)),
        ],
        out_shape=[
            jax.ShapeDtypeStruct((_N, _F), f32),
            jax.ShapeDtypeStruct((_N, _D), f32),
            jax.ShapeDtypeStruct((n_db, 1, 128), f32),
            jax.ShapeDtypeStruct((n_db, 1, 128), f32),
            jax.ShapeDtypeStruct((1, 128), f32),
        ],
        scratch_shapes=[pltpu.VMEM((_N, _D), f32)],
    )(jnp.reshape(tau, (1,)), acts, W_dec, xn, mu, sd, b_dec)

    l1n = jnp.sum(l1p[:, 0, 0]) / _N
    l0n = jnp.sum(l0p[:, 0, 0]) / _N
    l2 = l2p[0, 0] / (_N * _D)
    loss = l2 + jnp.float32(0.0)
    return sae, atk, loss, l2, _L1C * l1n, l0n, l1n


# speedup vs baseline: 66.2158x; 1.0070x over previous
import functools

import jax
import jax.numpy as jnp
from jax import lax
from jax.experimental import pallas as pl
from jax.experimental.pallas import tpu as pltpu
from jax.experimental.pallas import tpu_sc as plsc

_N = 2048
_D = 768
_F = 16384
_KTOT = 32 * _N
_L1C = 1e-4

_EB = 1024
_DB = 512
_NB = 8
_RR = 11

_BND = ((126 << 23) | (1 << 22), 127 << 23,
        (127 << 23) | (1 << 22), 128 << 23)

_NWK = 32
_ROWS_W = _N // _NWK
_CCAP = 32768
_SLAB = 8
_CW = 4096


def _prep_body(x_ref, xn_ref, mu_ref, sd_ref):
    x = x_ref[...]
    mu = jnp.mean(x, axis=1, keepdims=True)
    xc = x - mu
    m2 = jnp.mean(xc, axis=1, keepdims=True)
    var = jnp.sum((xc - m2) * (xc - m2), axis=1, keepdims=True) / (_D - 1)
    sd = jnp.sqrt(var)
    xn_ref[...] = xc / (sd + 1e-5)
    mu_ref[...] = jnp.broadcast_to(mu, (_N, 128))
    sd_ref[...] = jnp.broadcast_to(sd, (_N, 128))


def _enc_body(xn_ref, w_ref, b_ref, acts_ref, cnt_ref, cnt_sm):
    t = pl.program_id(0)
    nt = pl.num_programs(0)

    @pl.when(t == 0)
    def _():
        for i in range(len(_BND)):
            cnt_sm[i] = 0

    xc = xn_ref[...] - b_ref[...]
    acts = jax.lax.dot_general(xc, w_ref[...], (((1,), (0,)), ((), ())),
                               preferred_element_type=jnp.float32)
    acts = jnp.maximum(acts, 0.0)
    acts_ref[...] = acts
    bits = jax.lax.bitcast_convert_type(acts, jnp.int32)
    for i, b in enumerate(_BND):
        cnt_sm[i] = cnt_sm[i] + jnp.sum((bits >= b).astype(jnp.int32))

    @pl.when(t == nt - 1)
    def _():
        cnt_ref[...] = jnp.concatenate(
            [jnp.full((1, 128), cnt_sm[i], jnp.int32)
             for i in range(len(_BND))], axis=0)


def _compact_body(acts_hbm, lo_hbm, cand_hbm, cnt_hbm,
                  win_a, win_b, cand_v, lo_v, cnt_v, sem_a, sem_b):
    c = lax.axis_index("c")
    s = lax.axis_index("s")
    wid = s * 2 + c
    row0 = wid * _ROWS_W
    pltpu.sync_copy(lo_hbm, lo_v)
    lo = lo_v[...]

    def zb(i, _):
        cand_v[pl.ds(i * 16, 16)] = jnp.zeros((16,), jnp.float32)
        return 0

    lax.fori_loop(0, (_CCAP + 64) // 16, zb, 0)

    def src(w):
        s = jax.lax.div(w, 4)
        q = jax.lax.rem(w, 4)
        return acts_hbm.at[pl.ds(row0 + _SLAB * s, _SLAB),
                           pl.ds(_CW * q, _CW)]

    pltpu.async_copy(src(0), win_a, sem_a)

    iota16 = lax.iota(jnp.int32, 16)
    one16 = jnp.ones((16,), jnp.int32)
    zero16 = jnp.zeros((16,), jnp.int32)
    _UNR = 4
    _PLU = _CCAP // (16 * _UNR)

    def process(buf, cnts):
        def vb(i, cnts):
            base = i * (16 * _UNR)
            out = list(cnts)
            for r in range(_SLAB):
                for u in range(_UNR):
                    v = buf[r, pl.ds(base + 16 * u, 16)]
                    m = v >= lo
                    idx = (jnp.minimum(out[u], _PLU - 1) * (16 * _UNR)
                           + (u * 16) + iota16)
                    plsc.store_scatter(cand_v, [idx], v, mask=m)
                    out[u] = out[u] + jnp.where(m, one16, zero16)
            return tuple(out)
        return plsc.parallel_loop(0, _CW // (16 * _UNR), unroll=4,
                                  carry=tuple(cnts))(vb)

    nwin = (_ROWS_W // _SLAB) * (_F // _CW)

    def gbody(g, cnts):
        w0 = 2 * g
        pltpu.async_copy(src(w0 + 1), win_b, sem_b)
        pltpu.make_async_copy(src(w0), win_a, sem_a).wait()
        cnts = process(win_a, cnts)

        @pl.when(g < nwin // 2 - 1)
        def _():
            pltpu.async_copy(src(w0 + 2), win_a, sem_a)

        pltpu.make_async_copy(src(w0 + 1), win_b, sem_b).wait()
        cnts = process(win_b, cnts)
        return cnts

    cnts0 = tuple(jnp.zeros((16,), jnp.int32) for _ in range(_UNR))
    cnts = lax.fori_loop(0, nwin // 2, gbody, cnts0)
    tot = zero16
    for u in range(_UNR):
        tot = tot + jnp.minimum(cnts[u], _PLU)
    cnt_v[...] = tot
    pltpu.sync_copy(cnt_v, cnt_hbm.at[wid])
    pltpu.sync_copy(cand_v.at[pl.ds(0, _CCAP)], cand_hbm.at[wid])


def _refine_body(lob_ref, cand_ref, tau_ref):
    def rbody(r, lh):
        lo, hi = lh
        step = jnp.maximum(jax.lax.div(hi - lo, _NB + 1), 1)
        bits = jax.lax.bitcast_convert_type(cand_ref[...], jnp.int32)
        nlo, nhi = lo, hi
        for i in range(_NB):
            b = lo + step * (i + 1)
            cval = jnp.sum((bits >= b).astype(jnp.int32))
            ok = cval >= _KTOT
            nlo = jnp.where(ok, jnp.maximum(nlo, b), nlo)
            nhi = jnp.where(ok, nhi, jnp.minimum(nhi, b))
        return (nlo, nhi)

    lo, hi = lax.fori_loop(0, _RR, rbody, (lob_ref[0], lob_ref[1]))
    tau_ref[...] = jnp.full((8, 128), lo, jnp.int32)


def _dec_body(tau_ref, acts_ref, w_ref, xn_ref, mu_ref, sd_ref, b_ref,
              atk_ref, sae_ref, l1p_ref, l0p_ref, l2p_ref, acc):
    t = pl.program_id(0)
    nt = pl.num_programs(0)
    tau = tau_ref[0]
    a = acts_ref[...]
    m = a >= tau
    atk = jnp.where(m, a, 0.0)
    atk_ref[...] = atk
    l1p_ref[...] = jnp.full((1, 1, 128), jnp.sum(atk), jnp.float32)
    l0p_ref[...] = jnp.full((1, 1, 128), jnp.sum(m.astype(jnp.float32)),
                            jnp.float32)

    @pl.when(t == 0)
    def _():
        acc[...] = jnp.zeros_like(acc)

    acc[...] += jax.lax.dot_general(atk, w_ref[...], (((1,), (0,)), ((), ())),
                                    preferred_element_type=jnp.float32)

    @pl.when(t == nt - 1)
    def _():
        xr = acc[...] + b_ref[...]
        xn = xn_ref[...]
        d = xr - xn
        l2p_ref[...] = jnp.full((1, 128), jnp.sum(d * d), jnp.float32)
        sae_ref[...] = xr * sd_ref[...][:, 0:1] + mu_ref[...][:, 0:1]


@jax.jit
def kernel(x, W_enc, W_dec, b_dec):
    f32 = jnp.float32

    xn, mu, sd = pl.pallas_call(
        _prep_body,
        out_shape=[
            jax.ShapeDtypeStruct((_N, _D), f32),
            jax.ShapeDtypeStruct((_N, 128), f32),
            jax.ShapeDtypeStruct((_N, 128), f32),
        ],
    )(x)

    n_eb = _F // _EB
    acts, cntlad = pl.pallas_call(
        _enc_body,
        grid=(n_eb,),
        in_specs=[
            pl.BlockSpec((_N, _D), lambda i: (0, 0)),
            pl.BlockSpec((_D, _EB), lambda i: (0, i)),
            pl.BlockSpec((_D,), lambda i: (0,)),
        ],
        out_specs=[
            pl.BlockSpec((_N, _EB), lambda i: (0, i)),
            pl.BlockSpec((len(_BND), 128), lambda i: (0, 0)),
        ],
        out_shape=[
            jax.ShapeDtypeStruct((_N, _F), f32),
            jax.ShapeDtypeStruct((len(_BND), 128), jnp.int32),
        ],
        scratch_shapes=[pltpu.SMEM((len(_BND),), jnp.int32)],
    )(xn, W_enc, b_dec)

    cnts = cntlad[:, 0]
    bnd_arr = jnp.array(_BND, jnp.int32)
    nok = jnp.sum((cnts >= _KTOT).astype(jnp.int32))
    lo_bits = jnp.where(nok > 0, bnd_arr[jnp.maximum(nok - 1, 0)],
                        jnp.int32(1))
    hi_bits = jnp.where(nok < len(_BND),
                        bnd_arr[jnp.minimum(nok, len(_BND) - 1)],
                        jnp.int32(0x7F800000))
    lo_vec = jnp.full((16,), lax.bitcast_convert_type(lo_bits, f32), f32)

    mesh = plsc.VectorSubcoreMesh(core_axis_name="c", subcore_axis_name="s")
    compact = functools.partial(
        pl.kernel,
        mesh=mesh,
        compiler_params=pltpu.CompilerParams(needs_layout_passes=False),
        out_type=[
            jax.ShapeDtypeStruct((_NWK, _CCAP), f32),
            jax.ShapeDtypeStruct((_NWK, 16), jnp.int32),
        ],
        scratch_types=[
            pltpu.VMEM((_SLAB, _CW), f32),
            pltpu.VMEM((_SLAB, _CW), f32),
            pltpu.VMEM((_CCAP + 64,), f32),
            pltpu.VMEM((16,), f32),
            pltpu.VMEM((16,), jnp.int32),
            pltpu.SemaphoreType.DMA,
            pltpu.SemaphoreType.DMA,
        ],
    )(_compact_body)
    cand, scnt = compact(acts, lo_vec)

    lob = jnp.stack([lo_bits, hi_bits])
    taub = pl.pallas_call(
        _refine_body,
        in_specs=[
            pl.BlockSpec(memory_space=pltpu.SMEM),
            pl.BlockSpec((_NWK, _CCAP), lambda: (0, 0)),
        ],
        out_specs=pl.BlockSpec((8, 128), lambda: (0, 0)),
        out_shape=jax.ShapeDtypeStruct((8, 128), jnp.int32),
    )(lob, cand)

    c_total = jnp.sum(scnt)
    tau = jnp.where(c_total >= _KTOT,
                    lax.bitcast_convert_type(taub[0, 0], f32),
                    jnp.float32(0.0))

    n_db = _F // _DB
    atk, sae, l1p, l0p, l2p = pl.pallas_call(
        _dec_body,
        grid=(n_db,),
        in_specs=[
            pl.BlockSpec(memory_space=pltpu.SMEM),
            pl.BlockSpec((_N, _DB), lambda i: (0, i)),
            pl.BlockSpec((_DB, _D), lambda i: (i, 0)),
            pl.BlockSpec((_N, _D), lambda i: (0, 0)),
            pl.BlockSpec((_N, 128), lambda i: (0, 0)),
            pl.BlockSpec((_N, 128), lambda i: (0, 0)),
            pl.BlockSpec((_D,), lambda i: (0,)),
        ],
        out_specs=[
            pl.BlockSpec((_N, _DB), lambda i: (0, i)),
            pl.BlockSpec((_N, _D), lambda i: (0, 0)),
            pl.BlockSpec((1, 1, 128), lambda i: (i, 0, 0)),
            pl.BlockSpec((1, 1, 128), lambda i: (i, 0, 0)),
            pl.BlockSpec((1, 128), lambda i: (0, 0)),
        ],
        out_shape=[
            jax.ShapeDtypeStruct((_N, _F), f32),
            jax.ShapeDtypeStruct((_N, _D), f32),
            jax.ShapeDtypeStruct((n_db, 1, 128), f32),
            jax.ShapeDtypeStruct((n_db, 1, 128), f32),
            jax.ShapeDtypeStruct((1, 128), f32),
        ],
        scratch_shapes=[pltpu.VMEM((_N, _D), f32)],
    )(jnp.reshape(tau, (1,)), acts, W_dec, xn, mu, sd, b_dec)

    l1n = jnp.sum(l1p[:, 0, 0]) / _N
    l0n = jnp.sum(l0p[:, 0, 0]) / _N
    l2 = l2p[0, 0] / (_N * _D)
    loss = l2 + jnp.float32(0.0)
    return sae, atk, loss, l2, _L1C * l1n, l0n, l1n
